# baseline (speedup 1.0000x reference)
import functools

import jax
import jax.numpy as jnp
from jax import lax
from jax.experimental import pallas as pl
from jax.experimental.pallas import tpu as pltpu, tpu_sc as plsc

B = 512
SEQ = 336
PRED = 96
C = 7
POGT = 48
CAP = 100000
D = 128
K = 16
TEMP = 0.1
TRUST = 0.5
STEEP = 10.0

COLB = 4096
NSTEP = -(-CAP // COLB)
COLA = 4096
NSTEPA = -(-CAP // COLA)
CAPP = NSTEPA * COLA
SUB = 128
NB = CAPP // SUB
NEG = -1e30
BIG = 2 ** 30

NW = 32
SEGS = B * K
VROW = PRED * C


def _dense_body(pogt2_ref, wenc_ref, keys_ref, sims_ref, cand_ref, qn_ref,
                m_scr, qn_scr):
    j = pl.program_id(0)

    @pl.when(j == 0)
    def _init():
        q = jnp.dot(pogt2_ref[...], wenc_ref[...],
                    preferred_element_type=jnp.float32)
        qn = q / (jnp.sqrt(jnp.sum(q * q, axis=1, keepdims=True)) + 1e-8)
        qn_scr[...] = qn
        qn_ref[...] = qn

    kb = keys_ref[...]
    ss = jnp.sum(kb * kb, axis=1)
    inv = 1.0 / (jnp.sqrt(ss) + 1e-8)
    s = lax.dot_general(qn_scr[...].astype(jnp.bfloat16),
                        kb.astype(jnp.bfloat16), (((1,), (1,)), ((), ())),
                        preferred_element_type=jnp.float32)
    s = s * inv[None, :]
    col = j * COLA + lax.broadcasted_iota(jnp.int32, (B, COLA), 1)
    s = jnp.where(col < CAP, s, NEG)
    sims_ref[...] = s
    bmax = jnp.max(s.reshape(B, COLA // SUB, SUB), axis=2)
    mpad = jnp.concatenate(
        [bmax, jnp.full((B, SUB - COLA // SUB), NEG, jnp.float32)], axis=1)
    m_scr[:, pl.ds(j * SUB, SUB)] = mpad

    @pl.when(j == NSTEPA - 1)
    def _blk_topk():
        x = m_scr[...].reshape(B, NSTEPA, SUB)[:, :, :COLA // SUB]
        x = x.reshape(B, NB)
        it = lax.broadcasted_iota(jnp.int32, (B, NB), 1)
        brow = lax.broadcasted_iota(jnp.int32, (B, 1), 0)
        for t in range(K):
            m = jnp.max(x, axis=1, keepdims=True)
            cpos = jnp.where(x == m, it, BIG)
            blk = jnp.min(cpos, axis=1, keepdims=True)
            x = jnp.where(it == blk, NEG, x)
            cand_ref[:, pl.ds(t, 1)] = ((brow // 8) * NB + blk) * 8 + brow % 8


def _dense_stage(pogt2, W_enc, mem_keys):
    return pl.pallas_call(
        _dense_body,
        grid=(NSTEPA,),
        in_specs=[
            pl.BlockSpec((B, SEQ), lambda j: (0, 0)),
            pl.BlockSpec((SEQ, D), lambda j: (0, 0)),
            pl.BlockSpec((COLA, D), lambda j: (j, 0)),
        ],
        out_specs=[
            pl.BlockSpec((B, COLA), lambda j: (0, j)),
            pl.BlockSpec((B, K), lambda j: (0, 0)),
            pl.BlockSpec((B, D), lambda j: (0, 0)),
        ],
        out_shape=[
            jax.ShapeDtypeStruct((B, CAPP), jnp.float32),
            jax.ShapeDtypeStruct((B, K), jnp.int32),
            jax.ShapeDtypeStruct((B, D), jnp.float32),
        ],
        scratch_shapes=[
            pltpu.VMEM((B, NSTEPA * SUB), jnp.float32),
            pltpu.VMEM((B, D), jnp.float32),
        ],
    )(pogt2, W_enc, mem_keys)


@functools.lru_cache(maxsize=None)
def _make_sc_gather(n_idx, row_w, chunk):
    per_w = n_idx // NW
    nch = per_w // chunk
    mesh = plsc.VectorSubcoreMesh(core_axis_name="c", subcore_axis_name="s")

    @functools.partial(
        pl.kernel,
        out_type=jax.ShapeDtypeStruct((n_idx, row_w), jnp.float32),
        mesh=mesh,
        scratch_types=(
            [pltpu.VMEM((chunk,), jnp.int32)] * nch
            + [pltpu.VMEM((chunk, row_w), jnp.float32)] * nch
            + [pltpu.SemaphoreType.DMA]
        ),
    )
    def _g(table_hbm, idx_hbm, out_hbm, *scr):
        idx_vs, rows_vs, sem = scr[:nch], scr[nch:2 * nch], scr[-1]
        wid = lax.axis_index("s") * 2 + lax.axis_index("c")
        base = wid * per_w
        for h in range(nch):
            pltpu.sync_copy(idx_hbm.at[pl.ds(base + h * chunk, chunk)],
                            idx_vs[h])
        copies = [pltpu.async_copy(table_hbm.at[idx_vs[h]], rows_vs[h], sem)
                  for h in range(nch)]
        for h in range(nch):
            copies[h].wait()
            pltpu.sync_copy(rows_vs[h], out_hbm.at[pl.ds(base + h * chunk,
                                                         chunk)])

    return _g


def _corr_body(keys_ref, qn_ref, v3_ref, cand_ref, xt3_ref, wt_ref,
               bias_ref, out_ref, acc_scr, sc_scr):
    j = pl.program_id(0)

    @pl.when(j == 0)
    def _init():
        x = cand_ref[...]
        it = lax.broadcasted_iota(jnp.int32, (B, K * SUB), 1)
        tops = []
        for t in range(K):
            m = jnp.max(x, axis=1, keepdims=True)
            cpos = jnp.where(x == m, it, BIG)
            pos = jnp.min(cpos, axis=1, keepdims=True)
            x = jnp.where(it == pos, NEG, x)
            tops.append(m)
        top = jnp.concatenate(tops, axis=1)
        conf = top[:, 0:1]
        z = jnp.sum(jnp.exp((top - conf) / TEMP), axis=1, keepdims=True)
        gate = 1.0 / (1.0 + jnp.exp(-STEEP * (conf - TRUST)))
        sc_scr[:, 0:1] = conf
        sc_scr[:, 1:2] = tops[K - 1]
        sc_scr[:, 2:3] = gate / z
        bias = bias_ref[...]
        for c in range(C):
            acc_scr[pl.ds(c * PRED, PRED), :] = jnp.dot(
                wt_ref[...], xt3_ref[c],
                preferred_element_type=jnp.float32) + bias

    kb = keys_ref[...]
    ss = jnp.sum(kb * kb, axis=1)
    inv = 1.0 / (jnp.sqrt(ss) + 1e-8)
    s = lax.dot_general(qn_ref[...].astype(jnp.bfloat16),
                        kb.astype(jnp.bfloat16), (((1,), (1,)), ((), ())),
                        preferred_element_type=jnp.float32)
    s = s * inv[None, :]
    scol = j * COLB + lax.broadcasted_iota(jnp.int32, (B, COLB), 1)
    s = jnp.where(scol < CAP, s, NEG)
    conf = sc_scr[:, 0:1]
    w = jnp.exp((s - conf) / TEMP) * sc_scr[:, 2:3]
    w = jnp.where(s >= sc_scr[:, 1:2], w, 0.0)
    v = v3_ref[...].reshape(VROW, COLB)
    col = j * COLB + lax.broadcasted_iota(jnp.int32, (VROW, COLB), 1)
    v = jnp.where(col < CAP, v, 0.0)
    acc_scr[...] += lax.dot_general(
        v.astype(jnp.bfloat16), w.astype(jnp.bfloat16),
        (((1,), (1,)), ((), ())), preferred_element_type=jnp.float32)

    @pl.when(j == NSTEP - 1)
    def _emit():
        out_ref[...] = acc_scr[...]


def _corr_stage(mem_keys, qn, v3, cand, xt3, wt, bias2):
    return pl.pallas_call(
        _corr_body,
        grid=(NSTEP,),
        in_specs=[
            pl.BlockSpec((COLB, D), lambda j: (j, 0)),
            pl.BlockSpec((B, D), lambda j: (0, 0)),
            pl.BlockSpec((C, PRED, COLB), lambda j: (0, 0, j)),
            pl.BlockSpec((B, K * SUB), lambda j: (0, 0)),
            pl.BlockSpec((C, SEQ, B), lambda j: (0, 0, 0)),
            pl.BlockSpec((PRED, SEQ), lambda j: (0, 0)),
            pl.BlockSpec((PRED, 1), lambda j: (0, 0)),
        ],
        out_specs=pl.BlockSpec((VROW, B), lambda j: (0, 0)),
        out_shape=jax.ShapeDtypeStruct((VROW, B), jnp.float32),
        scratch_shapes=[
            pltpu.VMEM((VROW, B), jnp.float32),
            pltpu.VMEM((B, SUB), jnp.float32),
        ],
    )(mem_keys, qn, v3, cand, xt3, wt, bias2)


def kernel(x_enc, pogt, W_backbone, b_backbone, W_enc, mem_keys, mem_values):
    pogt2 = pogt.reshape(B, POGT * C)
    xt3 = jnp.transpose(x_enc, (2, 1, 0))
    v3 = jnp.transpose(mem_values, (2, 1, 0))
    wt = jnp.transpose(W_backbone, (1, 0))
    bias2 = b_backbone.reshape(PRED, 1)

    sims, cand_rows, qn = _dense_stage(pogt2, W_enc, mem_keys)

    simsr = (sims.reshape(B // 8, 8, NB, SUB)
             .transpose(0, 2, 1, 3).reshape(B * NB, SUB))
    cand = _make_sc_gather(SEGS, SUB, 128)(simsr, cand_rows.reshape(SEGS))

    out_t = _corr_stage(mem_keys, qn, v3, cand.reshape(B, K * SUB), xt3, wt,
                        bias2)
    return jnp.transpose(out_t.reshape(C, PRED, B), (2, 1, 0))


# speedup vs baseline: 1.0249x; 1.0249x over previous
import functools

import jax
import jax.numpy as jnp
from jax import lax
from jax.experimental import pallas as pl
from jax.experimental.pallas import tpu as pltpu, tpu_sc as plsc

B = 512
SEQ = 336
PRED = 96
C = 7
POGT = 48
CAP = 100000
D = 128
K = 16
TEMP = 0.1
TRUST = 0.5
STEEP = 10.0

COLB = 2048
NSTEP = -(-CAP // COLB)
COLA = 4096
NSTEPA = -(-CAP // COLA)
CAPP = NSTEPA * COLA
SUB = 128
NB = CAPP // SUB
NEG = -1e30
BIG = 2 ** 30

NW = 32
SEGS = B * K
VROW = PRED * C


def _dense_body(pogt2_ref, wenc_ref, keys_ref, sims_ref, cand_ref, qn_ref,
                m_scr, qn_scr):
    j = pl.program_id(0)

    @pl.when(j == 0)
    def _init():
        q = jnp.dot(pogt2_ref[...], wenc_ref[...],
                    preferred_element_type=jnp.float32)
        qn = q / (jnp.sqrt(jnp.sum(q * q, axis=1, keepdims=True)) + 1e-8)
        qn_scr[...] = qn
        qn_ref[...] = qn

    kb = keys_ref[...]
    ss = jnp.sum(kb * kb, axis=1)
    inv = 1.0 / (jnp.sqrt(ss) + 1e-8)
    s = lax.dot_general(qn_scr[...].astype(jnp.bfloat16),
                        kb.astype(jnp.bfloat16), (((1,), (1,)), ((), ())),
                        preferred_element_type=jnp.float32)
    s = s * inv[None, :]
    col = j * COLA + lax.broadcasted_iota(jnp.int32, (B, COLA), 1)
    s = jnp.where(col < CAP, s, NEG)
    sims_ref[...] = s
    bmax = jnp.max(s.reshape(B, COLA // SUB, SUB), axis=2)
    mpad = jnp.concatenate(
        [bmax, jnp.full((B, SUB - COLA // SUB), NEG, jnp.float32)], axis=1)
    m_scr[:, pl.ds(j * SUB, SUB)] = mpad

    @pl.when(j == NSTEPA - 1)
    def _blk_topk():
        x = m_scr[...].reshape(B, NSTEPA, SUB)[:, :, :COLA // SUB]
        x = x.reshape(B, NB)
        it = lax.broadcasted_iota(jnp.int32, (B, NB), 1)
        brow = lax.broadcasted_iota(jnp.int32, (B, 1), 0)
        for t in range(K):
            m = jnp.max(x, axis=1, keepdims=True)
            cpos = jnp.where(x == m, it, BIG)
            blk = jnp.min(cpos, axis=1, keepdims=True)
            x = jnp.where(it == blk, NEG, x)
            cand_ref[:, pl.ds(t, 1)] = ((brow // 8) * NB + blk) * 8 + brow % 8


def _dense_stage(pogt2, W_enc, mem_keys):
    return pl.pallas_call(
        _dense_body,
        grid=(NSTEPA,),
        in_specs=[
            pl.BlockSpec((B, SEQ), lambda j: (0, 0)),
            pl.BlockSpec((SEQ, D), lambda j: (0, 0)),
            pl.BlockSpec((COLA, D), lambda j: (j, 0)),
        ],
        out_specs=[
            pl.BlockSpec((B, COLA), lambda j: (0, j)),
            pl.BlockSpec((B, K), lambda j: (0, 0)),
            pl.BlockSpec((B, D), lambda j: (0, 0)),
        ],
        out_shape=[
            jax.ShapeDtypeStruct((B, CAPP), jnp.float32),
            jax.ShapeDtypeStruct((B, K), jnp.int32),
            jax.ShapeDtypeStruct((B, D), jnp.float32),
        ],
        scratch_shapes=[
            pltpu.VMEM((B, NSTEPA * SUB), jnp.float32),
            pltpu.VMEM((B, D), jnp.float32),
        ],
    )(pogt2, W_enc, mem_keys)


@functools.lru_cache(maxsize=None)
def _make_sc_gather(n_idx, row_w, chunk):
    per_w = n_idx // NW
    nch = per_w // chunk
    mesh = plsc.VectorSubcoreMesh(core_axis_name="c", subcore_axis_name="s")

    @functools.partial(
        pl.kernel,
        out_type=jax.ShapeDtypeStruct((n_idx, row_w), jnp.float32),
        mesh=mesh,
        scratch_types=(
            [pltpu.VMEM((chunk,), jnp.int32)] * nch
            + [pltpu.VMEM((chunk, row_w), jnp.float32)] * nch
            + [pltpu.SemaphoreType.DMA]
        ),
    )
    def _g(table_hbm, idx_hbm, out_hbm, *scr):
        idx_vs, rows_vs, sem = scr[:nch], scr[nch:2 * nch], scr[-1]
        wid = lax.axis_index("s") * 2 + lax.axis_index("c")
        base = wid * per_w
        for h in range(nch):
            pltpu.sync_copy(idx_hbm.at[pl.ds(base + h * chunk, chunk)],
                            idx_vs[h])
        copies = [pltpu.async_copy(table_hbm.at[idx_vs[h]], rows_vs[h], sem)
                  for h in range(nch)]
        for h in range(nch):
            copies[h].wait()
            pltpu.sync_copy(rows_vs[h], out_hbm.at[pl.ds(base + h * chunk,
                                                         chunk)])

    return _g


def _corr_body(keys_ref, qn_ref, v3_ref, cand_ref, xt3_ref, wt_ref,
               bias_ref, out_ref, acc_scr, sc_scr):
    j = pl.program_id(0)

    @pl.when(j == 0)
    def _init():
        x = cand_ref[...]
        it = lax.broadcasted_iota(jnp.int32, (B, K * SUB), 1)
        tops = []
        for t in range(K):
            m = jnp.max(x, axis=1, keepdims=True)
            cpos = jnp.where(x == m, it, BIG)
            pos = jnp.min(cpos, axis=1, keepdims=True)
            x = jnp.where(it == pos, NEG, x)
            tops.append(m)
        top = jnp.concatenate(tops, axis=1)
        conf = top[:, 0:1]
        z = jnp.sum(jnp.exp((top - conf) / TEMP), axis=1, keepdims=True)
        gate = 1.0 / (1.0 + jnp.exp(-STEEP * (conf - TRUST)))
        sc_scr[:, 0:1] = conf
        sc_scr[:, 1:2] = tops[K - 1]
        sc_scr[:, 2:3] = gate / z
        bias = bias_ref[...]
        for c in range(C):
            acc_scr[pl.ds(c * PRED, PRED), :] = jnp.dot(
                wt_ref[...], xt3_ref[c],
                preferred_element_type=jnp.float32) + bias

    kb = keys_ref[...]
    ss = jnp.sum(kb * kb, axis=1)
    inv = 1.0 / (jnp.sqrt(ss) + 1e-8)
    s = lax.dot_general(qn_ref[...].astype(jnp.bfloat16),
                        kb.astype(jnp.bfloat16), (((1,), (1,)), ((), ())),
                        preferred_element_type=jnp.float32)
    s = s * inv[None, :]
    scol = j * COLB + lax.broadcasted_iota(jnp.int32, (B, COLB), 1)
    s = jnp.where(scol < CAP, s, NEG)
    conf = sc_scr[:, 0:1]
    w = jnp.exp((s - conf) / TEMP) * sc_scr[:, 2:3]
    w = jnp.where(s >= sc_scr[:, 1:2], w, 0.0)
    v = v3_ref[...].reshape(VROW, COLB)
    col = j * COLB + lax.broadcasted_iota(jnp.int32, (VROW, COLB), 1)
    v = jnp.where(col < CAP, v, 0.0)
    acc_scr[...] += lax.dot_general(
        v.astype(jnp.bfloat16), w.astype(jnp.bfloat16),
        (((1,), (1,)), ((), ())), preferred_element_type=jnp.float32)

    @pl.when(j == NSTEP - 1)
    def _emit():
        out_ref[...] = acc_scr[...]


def _corr_stage(mem_keys, qn, v3, cand, xt3, wt, bias2):
    return pl.pallas_call(
        _corr_body,
        grid=(NSTEP,),
        in_specs=[
            pl.BlockSpec((COLB, D), lambda j: (j, 0)),
            pl.BlockSpec((B, D), lambda j: (0, 0)),
            pl.BlockSpec((C, PRED, COLB), lambda j: (0, 0, j)),
            pl.BlockSpec((B, K * SUB), lambda j: (0, 0)),
            pl.BlockSpec((C, SEQ, B), lambda j: (0, 0, 0)),
            pl.BlockSpec((PRED, SEQ), lambda j: (0, 0)),
            pl.BlockSpec((PRED, 1), lambda j: (0, 0)),
        ],
        out_specs=pl.BlockSpec((VROW, B), lambda j: (0, 0)),
        out_shape=jax.ShapeDtypeStruct((VROW, B), jnp.float32),
        scratch_shapes=[
            pltpu.VMEM((VROW, B), jnp.float32),
            pltpu.VMEM((B, SUB), jnp.float32),
        ],
    )(mem_keys, qn, v3, cand, xt3, wt, bias2)


def kernel(x_enc, pogt, W_backbone, b_backbone, W_enc, mem_keys, mem_values):
    pogt2 = pogt.reshape(B, POGT * C)
    xt3 = jnp.transpose(x_enc, (2, 1, 0))
    v3 = jnp.transpose(mem_values, (2, 1, 0))
    wt = jnp.transpose(W_backbone, (1, 0))
    bias2 = b_backbone.reshape(PRED, 1)

    sims, cand_rows, qn = _dense_stage(pogt2, W_enc, mem_keys)

    simsr = (sims.reshape(B // 8, 8, NB, SUB)
             .transpose(0, 2, 1, 3).reshape(B * NB, SUB))
    cand = _make_sc_gather(SEGS, SUB, 128)(simsr, cand_rows.reshape(SEGS))

    out_t = _corr_stage(mem_keys, qn, v3, cand.reshape(B, K * SUB), xt3, wt,
                        bias2)
    return jnp.transpose(out_t.reshape(C, PRED, B), (2, 1, 0))
